# Initial kernel scaffold; baseline (speedup 1.0000x reference)
import jax
import jax.numpy as jnp
from jax.experimental import pallas as pl


def kernel(task_embedding, edge_attr, params, edge_index):
    raise NotImplementedError("write your pallas kernel here")


# speedup vs baseline: 31.5739x; 31.5739x over previous
import functools

import jax
import jax.numpy as jnp
from jax import lax
from jax.experimental import pallas as pl
from jax.experimental.pallas import tpu as pltpu
from jax.experimental.pallas import tpu_sc as plsc

NC = 2
NS = 16
LANES = 16
NW = NC * NS
K = 400


def _make_edge_pass(np_, c, e):
    ew = e // NW
    nchunk = ew // K
    stripe = np_ // NS
    mesh = plsc.VectorSubcoreMesh(core_axis_name="c", subcore_axis_name="s",
                                  num_cores=NC, num_subcores=NS)

    @functools.partial(
        pl.kernel,
        out_type=(jax.ShapeDtypeStruct((NC, np_, c), jnp.float32),
                  jax.ShapeDtypeStruct((NC, np_), jnp.float32)),
        mesh=mesh,
        scratch_types=(
            pltpu.VMEM((np_,), jnp.float32),
            pltpu.VMEM((np_,), jnp.float32),
            pltpu.VMEM((nchunk, K), jnp.int32),
            pltpu.VMEM((nchunk, K), jnp.int32),
            pltpu.VMEM((nchunk, K), jnp.float32),
            pltpu.VMEM((K,), jnp.float32),
            pltpu.VMEM((K, c), jnp.float32),
            pltpu.VMEM((stripe,), jnp.float32),
            pltpu.VMEM_SHARED((np_, c), jnp.float32),
            pltpu.VMEM_SHARED((np_,), jnp.float32),
            pltpu.SemaphoreType.DMA,
        ),
        compiler_params=pltpu.CompilerParams(needs_layout_passes=False,
                                             use_tc_tiling_on_sc=False),
    )
    def ek(asrc_h, adst_h, aedge_h, hs_h, src_h, dst_h,
           out_h, den_h,
           asrc_v, adst_v, src_v, dst_v, aedge_v, ex_v, rows_v, zden_v,
           out_sh, den_sh, sem):
        ci = lax.axis_index("c")
        si = lax.axis_index("s")
        wid = ci * NS + si

        pltpu.sync_copy(asrc_h, asrc_v)
        pltpu.sync_copy(adst_h, adst_v)
        pltpu.sync_copy(src_h.at[wid], src_v)
        pltpu.sync_copy(dst_h.at[wid], dst_v)
        pltpu.sync_copy(aedge_h.at[wid], aedge_v)

        z16 = jnp.zeros((LANES,), jnp.float32)

        def zrow(i, carry):
            for q in range(c // LANES):
                rows_v[i, pl.ds(q * LANES, LANES)] = z16
            return carry
        lax.fori_loop(0, K, zrow, 0)

        def zden(i, carry):
            zden_v[pl.ds(pl.multiple_of(i * LANES, LANES), LANES)] = z16
            return carry
        lax.fori_loop(0, stripe // LANES, zden, 0)

        base = si * stripe
        pltpu.sync_copy(rows_v, out_sh.at[pl.ds(base, K)])
        pltpu.sync_copy(rows_v.at[pl.ds(0, stripe - K)],
                        out_sh.at[pl.ds(base + K, stripe - K)])
        pltpu.sync_copy(zden_v, den_sh.at[pl.ds(base, stripe)])
        plsc.subcore_barrier()

        def chunk(k, carry):
            gat = pltpu.async_copy(hs_h.at[src_v.at[k]], rows_v, sem)

            def jbody(j, cc):
                o = pl.multiple_of(j * LANES, LANES)
                s16 = src_v[k, pl.ds(o, LANES)]
                d16 = dst_v[k, pl.ds(o, LANES)]
                a = (plsc.load_gather(asrc_v, [s16])
                     + plsc.load_gather(adst_v, [d16])
                     + aedge_v[k, pl.ds(o, LANES)])
                a = jnp.where(a >= 0.0, a, 0.2 * a)
                ex_v[pl.ds(o, LANES)] = jnp.exp(a)
                return cc
            lax.fori_loop(0, K // LANES, jbody, 0)
            gat.wait()

            def sbody(j, cc):
                o = pl.multiple_of(j * LANES, LANES)
                for l in range(LANES):
                    scale = plsc.load_gather(
                        ex_v, [jnp.full((LANES,), o + l, jnp.int32)])
                    for q in range(c // LANES):
                        col = pl.ds(q * LANES, LANES)
                        rows_v[o + l, col] = rows_v[o + l, col] * scale
                return cc
            lax.fori_loop(0, K // LANES, sbody, 0)

            pltpu.sync_copy(rows_v, out_sh.at[dst_v.at[k]], add=True)
            pltpu.sync_copy(ex_v, den_sh.at[dst_v.at[k]], add=True)
            return carry
        lax.fori_loop(0, nchunk, chunk, 0)

        plsc.subcore_barrier()
        pltpu.sync_copy(out_sh.at[pl.ds(base, stripe)],
                        out_h.at[ci, pl.ds(base, stripe)])
        pltpu.sync_copy(den_sh.at[pl.ds(base, stripe)],
                        den_h.at[ci, pl.ds(base, stripe)])

    return ek


def _phase_a(x_pad, wsrc1, wres1, asw1, wdv1, wsrc2, wres2, asw2, wdv2):
    np_, din = x_pad.shape
    c = wsrc1.shape[1]
    blk = 1024
    nb = np_ // blk

    def body(x_ref, ws1, wr1, a1, d1, ws2, wr2, a2, d2,
             hs1_ref, res1_ref, as1_ref, ad1_ref,
             hs2_ref, res2_ref, as2_ref, ad2_ref):
        x = x_ref[...]
        for ws, wr, aw, wd, hs_ref, res_ref, as_ref, ad_ref in (
                (ws1, wr1, a1, d1, hs1_ref, res1_ref, as1_ref, ad1_ref),
                (ws2, wr2, a2, d2, hs2_ref, res2_ref, as2_ref, ad2_ref)):
            hs = jnp.dot(x, ws[...], preferred_element_type=jnp.float32)
            hs_ref[...] = hs
            res_ref[...] = jnp.dot(x, wr[...], preferred_element_type=jnp.float32)
            as_ref[0, 0, :] = jnp.sum(hs * aw[...], axis=1)
            ad_ref[0, 0, :] = jnp.sum(x * wd[...], axis=1)

    full = lambda s: pl.BlockSpec(s, lambda i: (0,) * len(s))
    out = pl.pallas_call(
        body,
        grid=(nb,),
        in_specs=[pl.BlockSpec((blk, din), lambda i: (i, 0)),
                  full((din, c)), full((din, c)), full((1, c)), full((1, din)),
                  full((din, c)), full((din, c)), full((1, c)), full((1, din))],
        out_specs=[pl.BlockSpec((blk, c), lambda i: (i, 0)),
                   pl.BlockSpec((blk, c), lambda i: (i, 0)),
                   pl.BlockSpec((1, 1, blk), lambda i: (i, 0, 0)),
                   pl.BlockSpec((1, 1, blk), lambda i: (i, 0, 0))] * 2,
        out_shape=[jax.ShapeDtypeStruct((np_, c), jnp.float32),
                   jax.ShapeDtypeStruct((np_, c), jnp.float32),
                   jax.ShapeDtypeStruct((nb, 1, blk), jnp.float32),
                   jax.ShapeDtypeStruct((nb, 1, blk), jnp.float32)] * 2,
    )(x_pad, wsrc1, wres1, asw1.reshape(1, c), wdv1.reshape(1, din),
      wsrc2, wres2, asw2.reshape(1, c), wdv2.reshape(1, din))
    return out


def _phase_edge_logits(ea_t8, w84):
    _, e = ea_t8.shape
    blk = 2000
    nb = e // blk

    def body(ea_ref, w_ref, o0, o1, o2, o3):
        ea = ea_ref[...][0]
        w = w_ref[...]
        for d, ref in enumerate((o0, o1, o2, o3)):
            ref[0, 0, :] = jnp.sum(ea * w[:, d:d + 1], axis=0)

    out = pl.pallas_call(
        body,
        grid=(nb,),
        in_specs=[pl.BlockSpec((1, 8, blk), lambda i: (i, 0, 0)),
                  pl.BlockSpec((8, 4), lambda i: (0, 0))],
        out_specs=[pl.BlockSpec((1, 1, blk), lambda i: (i, 0, 0))] * 4,
        out_shape=[jax.ShapeDtypeStruct((nb, 1, blk), jnp.float32)] * 4,
    )(ea_t8.reshape(8, nb, blk).transpose(1, 0, 2), w84)
    return [o.reshape(e) for o in out]


def _phase_c(u, den, res, g, b, bias, wsrc, wres, asw, wdv):
    nc_, np_, c = u.shape
    blk = 1024
    nb = np_ // blk

    def body(u_ref, den_ref, res_ref, g_ref, b_ref, bias_ref,
             ws_ref, wr_ref, aw_ref, wd_ref,
             hs_ref, res2_ref, as_ref, ad_ref):
        uu = u_ref[0] + u_ref[1]
        dd = den_ref[0, 0, 0] + den_ref[1, 0, 0]
        h = uu / (dd + 1e-16)[:, None] + res_ref[...] + bias_ref[...]
        m = jnp.mean(h, axis=-1, keepdims=True)
        v = jnp.mean((h - m) ** 2, axis=-1, keepdims=True)
        h = (h - m) / jnp.sqrt(v + 1e-5) * g_ref[...] + b_ref[...]
        h = jnp.where(h >= 0.0, h, 0.01 * h)
        hs = jnp.dot(h, ws_ref[...], preferred_element_type=jnp.float32)
        hs_ref[...] = hs
        res2_ref[...] = jnp.dot(h, wr_ref[...], preferred_element_type=jnp.float32)
        as_ref[0, 0, :] = jnp.sum(hs * aw_ref[...], axis=1)
        ad_ref[0, 0, :] = jnp.sum(h * wd_ref[...], axis=1)

    full = lambda s: pl.BlockSpec(s, lambda i: (0,) * len(s))
    return pl.pallas_call(
        body,
        grid=(nb,),
        in_specs=[pl.BlockSpec((NC, blk, c), lambda i: (0, i, 0)),
                  pl.BlockSpec((NC, 1, 1, blk), lambda i: (0, i, 0, 0)),
                  pl.BlockSpec((blk, c), lambda i: (i, 0)),
                  full((1, c)), full((1, c)), full((1, c)),
                  full((c, c)), full((c, c)), full((1, c)), full((1, c))],
        out_specs=[pl.BlockSpec((blk, c), lambda i: (i, 0)),
                   pl.BlockSpec((blk, c), lambda i: (i, 0)),
                   pl.BlockSpec((1, 1, blk), lambda i: (i, 0, 0)),
                   pl.BlockSpec((1, 1, blk), lambda i: (i, 0, 0))],
        out_shape=[jax.ShapeDtypeStruct((np_, c), jnp.float32),
                   jax.ShapeDtypeStruct((np_, c), jnp.float32),
                   jax.ShapeDtypeStruct((nb, 1, blk), jnp.float32),
                   jax.ShapeDtypeStruct((nb, 1, blk), jnp.float32)],
    )(u, den.reshape(NC, nb, 1, blk), res, g.reshape(1, c), b.reshape(1, c),
      bias.reshape(1, c), wsrc, wres, asw.reshape(1, c), wdv.reshape(1, c))


def _phase_e(u1, den1, res1, bias1, u2, den2, res2, bias2):
    nc_, np_, c = u1.shape
    blk = 1024
    nb = np_ // blk

    def body(u1_ref, d1_ref, r1_ref, b1_ref, u2_ref, d2_ref, r2_ref, b2_ref,
             out_ref):
        hs = []
        for u_ref, d_ref, r_ref, b_ref in ((u1_ref, d1_ref, r1_ref, b1_ref),
                                           (u2_ref, d2_ref, r2_ref, b2_ref)):
            uu = u_ref[0] + u_ref[1]
            dd = d_ref[0, 0, 0] + d_ref[1, 0, 0]
            hs.append(uu / (dd + 1e-16)[:, None] + r_ref[...] + b_ref[...])
        out_ref[...] = jnp.concatenate(hs, axis=1)

    full = lambda s: pl.BlockSpec(s, lambda i: (0,) * len(s))
    per_dir = [pl.BlockSpec((NC, blk, c), lambda i: (0, i, 0)),
               pl.BlockSpec((NC, 1, 1, blk), lambda i: (0, i, 0, 0)),
               pl.BlockSpec((blk, c), lambda i: (i, 0)),
               full((1, c))]
    return pl.pallas_call(
        body,
        grid=(nb,),
        in_specs=per_dir * 2,
        out_specs=pl.BlockSpec((blk, 2 * c), lambda i: (i, 0)),
        out_shape=jax.ShapeDtypeStruct((np_, 2 * c), jnp.float32),
    )(u1, den1.reshape(NC, nb, 1, blk), res1, bias1.reshape(1, c),
      u2, den2.reshape(NC, nb, 1, blk), res2, bias2.reshape(1, c))


def kernel(task_embedding, edge_attr, params, edge_index):
    n, din = task_embedding.shape
    e, edim = edge_attr.shape
    c = params["dep1"]["att_src"].shape[1]
    np_ = 10240

    x_pad = jnp.pad(task_embedding, ((0, np_ - n), (0, 0)))

    def _vecs(p):
        return (p["att_src"][0],
                p["W_dst"] @ p["att_dst"][0],
                p["W_edge"] @ p["att_edge"][0])

    as1d, wd1d, we1d = _vecs(params["dep1"])
    as1n, wd1n, we1n = _vecs(params["dan1"])
    as2d, wd2d, we2d = _vecs(params["dep2"])
    as2n, wd2n, we2n = _vecs(params["dan2"])

    ea_t8 = jnp.pad(edge_attr.T, ((0, 8 - edim), (0, 0)))
    w84 = jnp.pad(jnp.stack([we1d, we1n, we2d, we2n], axis=1),
                  ((0, 8 - edim), (0, 0)))
    ae1d, ae1n, ae2d, ae2n = _phase_edge_logits(ea_t8, w84)

    ew = e // NW
    nchunk = ew // K
    src_dep = edge_index[0].reshape(NW, nchunk, K)
    dst_dep = edge_index[1].reshape(NW, nchunk, K)
    src_dan, dst_dan = dst_dep, src_dep

    edge_pass = _make_edge_pass(np_, c, e)

    hs1d, res1d, a_s1d, a_d1d, hs1n, res1n, a_s1n, a_d1n = _phase_a(
        x_pad,
        params["dep1"]["W_src"], params["dep1"]["W_res"], as1d, wd1d,
        params["dan1"]["W_src"], params["dan1"]["W_res"], as1n, wd1n)

    u1d, den1d = edge_pass(a_s1d.reshape(np_), a_d1d.reshape(np_),
                           ae1d.reshape(NW, nchunk, K), hs1d,
                           src_dep, dst_dep)
    u1n, den1n = edge_pass(a_s1n.reshape(np_), a_d1n.reshape(np_),
                           ae1n.reshape(NW, nchunk, K), hs1n,
                           src_dan, dst_dan)

    hs2d, res2d, a_s2d, a_d2d = _phase_c(
        u1d, den1d, res1d, params["ln_dep_g"], params["ln_dep_b"],
        params["dep1"]["bias"], params["dep2"]["W_src"],
        params["dep2"]["W_res"], as2d, wd2d)
    hs2n, res2n, a_s2n, a_d2n = _phase_c(
        u1n, den1n, res1n, params["ln_dan_g"], params["ln_dan_b"],
        params["dan1"]["bias"], params["dan2"]["W_src"],
        params["dan2"]["W_res"], as2n, wd2n)

    u2d, den2d = edge_pass(a_s2d.reshape(np_), a_d2d.reshape(np_),
                           ae2d.reshape(NW, nchunk, K), hs2d,
                           src_dep, dst_dep)
    u2n, den2n = edge_pass(a_s2n.reshape(np_), a_d2n.reshape(np_),
                           ae2n.reshape(NW, nchunk, K), hs2n,
                           src_dan, dst_dan)

    out = _phase_e(u2d, den2d, res2d, params["dep2"]["bias"],
                   u2n, den2n, res2n, params["dan2"]["bias"])
    return out[:n]


# speedup vs baseline: 34.8916x; 1.1051x over previous
import functools

import jax
import jax.numpy as jnp
from jax import lax
from jax.experimental import pallas as pl
from jax.experimental.pallas import tpu as pltpu
from jax.experimental.pallas import tpu_sc as plsc

NC = 2
NS = 16
LANES = 16
K = 80


def _make_edge_pass(np_, c, e):
    ew = e // NS
    nchunk = ew // K
    stripe = np_ // NS
    nb = 5
    ngrp = nchunk // nb
    assert ngrp * nb == nchunk
    mesh = plsc.VectorSubcoreMesh(core_axis_name="c", subcore_axis_name="s",
                                  num_cores=NC, num_subcores=NS)

    @functools.partial(
        pl.kernel,
        out_type=(jax.ShapeDtypeStruct((NC, np_, c), jnp.float32),
                  jax.ShapeDtypeStruct((NC, np_), jnp.float32)),
        mesh=mesh,
        scratch_types=(
            pltpu.VMEM((np_,), jnp.float32),
            pltpu.VMEM((np_,), jnp.float32),
            pltpu.VMEM((nchunk, K), jnp.int32),
            pltpu.VMEM((nchunk, K), jnp.float32),
            *([pltpu.VMEM((K,), jnp.int32)] * nb),
            *([pltpu.VMEM((K,), jnp.int32)] * nb),
            *([pltpu.VMEM((K,), jnp.float32)] * nb),
            *([pltpu.VMEM((K, c), jnp.float32)] * nb),
            pltpu.VMEM((stripe,), jnp.float32),
            pltpu.VMEM_SHARED((np_, c), jnp.float32),
            pltpu.VMEM_SHARED((np_,), jnp.float32),
            *([pltpu.SemaphoreType.DMA] * (2 * nb)),
        ),
        compiler_params=pltpu.CompilerParams(needs_layout_passes=False,
                                             use_tc_tiling_on_sc=False),
    )
    def ek(asrc_h, adst_h, aedge_h, hs_h, sd_h,
           out_h, den_h,
           asrc_v, adst_v, sd_v, aedge_v, *tail):
        srcb = tail[:nb]
        dstb = tail[nb:2 * nb]
        exs = tail[2 * nb:3 * nb]
        rows = tail[3 * nb:4 * nb]
        zden_v = tail[4 * nb]
        out_sh = tail[4 * nb + 1]
        den_sh = tail[4 * nb + 2]
        gsem = tail[4 * nb + 3:5 * nb + 3]
        ssem = tail[5 * nb + 3:6 * nb + 3]
        ci = lax.axis_index("c")
        si = lax.axis_index("s")

        pltpu.sync_copy(asrc_h.at[ci], asrc_v)
        pltpu.sync_copy(adst_h.at[ci], adst_v)
        pltpu.sync_copy(sd_h.at[ci, si], sd_v)
        pltpu.sync_copy(aedge_h.at[ci, si], aedge_v)

        z16 = jnp.zeros((LANES,), jnp.float32)

        def zrow(i, carry):
            for q in range(c // LANES):
                rows[0][i, pl.ds(q * LANES, LANES)] = z16
            return carry
        lax.fori_loop(0, K, zrow, 0)

        def zden(i, carry):
            zden_v[pl.ds(pl.multiple_of(i * LANES, LANES), LANES)] = z16
            return carry
        lax.fori_loop(0, stripe // LANES, zden, 0)

        base = si * stripe
        for i in range(stripe // K):
            pltpu.sync_copy(rows[0], out_sh.at[pl.ds(base + i * K, K)])
        pltpu.sync_copy(zden_v, den_sh.at[pl.ds(base, stripe)])
        plsc.subcore_barrier()

        def issue_gather(k, b):
            for q in range(K // LANES):
                col = pl.ds(pl.multiple_of(q * LANES, LANES), LANES)
                v = sd_v[k, col]
                dstb[b][col] = lax.shift_right_logical(v, 14)
                srcb[b][col] = lax.bitwise_and(v, 16383)
            pltpu.async_copy(hs_h.at[ci].at[srcb[b]], rows[b], gsem[b])

        def wait_gather(b):
            pltpu.make_async_copy(hs_h.at[ci].at[srcb[b]], rows[b],
                                  gsem[b]).wait()

        def issue_scatter(k, b):
            pltpu.async_copy(rows[b], out_sh.at[dstb[b]], ssem[b],
                             add=True)
            pltpu.async_copy(exs[b], den_sh.at[dstb[b]], ssem[b],
                             add=True)

        def wait_scatter(b):
            pltpu.make_async_copy(rows[b], out_sh.at[dstb[b]],
                                  ssem[b]).wait()
            pltpu.make_async_copy(exs[b], den_sh.at[dstb[b]],
                                  ssem[b]).wait()

        def compute_ex(k, b, exb):
            def jbody(j, cc):
                o = pl.multiple_of(j * LANES, LANES)
                s16 = srcb[b][pl.ds(o, LANES)]
                d16 = dstb[b][pl.ds(o, LANES)]
                a = (plsc.load_gather(asrc_v, [s16])
                     + plsc.load_gather(adst_v, [d16])
                     + aedge_v[k, pl.ds(o, LANES)])
                a = jnp.where(a >= 0.0, a, 0.2 * a)
                exb[pl.ds(o, LANES)] = jnp.exp(a)
                return cc
            lax.fori_loop(0, K // LANES, jbody, 0)

        def scale_rows(rb, exb):
            def sbody(j, cc):
                o = pl.multiple_of(j * LANES, LANES)
                for l in range(LANES):
                    sc = plsc.load_gather(
                        exb, [jnp.full((LANES,), o + l, jnp.int32)])
                    for q in range(c // LANES):
                        col = pl.ds(q * LANES, LANES)
                        rb[o + l, col] = rb[o + l, col] * sc
                return cc
            lax.fori_loop(0, K // LANES, sbody, 0)

        issue_gather(0, 0)
        issue_gather(1, 1)

        def group(g, carry):
            for b in range(nb):
                k = g * nb + b
                b2 = (b + 2) % nb

                @pl.when(k >= nb - 2)
                def _():
                    wait_scatter(b2)

                @pl.when(k + 2 < nchunk)
                def _():
                    issue_gather(k + 2, b2)
                compute_ex(k, b, exs[b])
                wait_gather(b)
                scale_rows(rows[b], exs[b])
                issue_scatter(k, b)
            return carry
        lax.fori_loop(0, ngrp, group, 0)

        for j in range(nb - 2, 0, -1):
            wait_scatter((nchunk - j) % nb)

        plsc.subcore_barrier()
        pltpu.sync_copy(out_sh.at[pl.ds(base, stripe)],
                        out_h.at[ci, pl.ds(base, stripe)])
        pltpu.sync_copy(den_sh.at[pl.ds(base, stripe)],
                        den_h.at[ci, pl.ds(base, stripe)])

    return ek


def _phase_a(x_pad, wsrc1, wres1, asw1, wdv1, wsrc2, wres2, asw2, wdv2):
    np_, din = x_pad.shape
    c = wsrc1.shape[1]
    blk = 1024
    nb = np_ // blk

    def body(x_ref, ws1, wr1, a1, d1, ws2, wr2, a2, d2,
             hs1_ref, res1_ref, as1_ref, ad1_ref,
             hs2_ref, res2_ref, as2_ref, ad2_ref):
        x = x_ref[...]
        for ws, wr, aw, wd, hs_ref, res_ref, as_ref, ad_ref in (
                (ws1, wr1, a1, d1, hs1_ref, res1_ref, as1_ref, ad1_ref),
                (ws2, wr2, a2, d2, hs2_ref, res2_ref, as2_ref, ad2_ref)):
            hs = jnp.dot(x, ws[...], preferred_element_type=jnp.float32)
            hs_ref[...] = hs
            res_ref[...] = jnp.dot(x, wr[...], preferred_element_type=jnp.float32)
            as_ref[0, 0, :] = jnp.sum(hs * aw[...], axis=1)
            ad_ref[0, 0, :] = jnp.sum(x * wd[...], axis=1)

    full = lambda s: pl.BlockSpec(s, lambda i: (0,) * len(s))
    out = pl.pallas_call(
        body,
        grid=(nb,),
        in_specs=[pl.BlockSpec((blk, din), lambda i: (i, 0)),
                  full((din, c)), full((din, c)), full((1, c)), full((1, din)),
                  full((din, c)), full((din, c)), full((1, c)), full((1, din))],
        out_specs=[pl.BlockSpec((blk, c), lambda i: (i, 0)),
                   pl.BlockSpec((blk, c), lambda i: (i, 0)),
                   pl.BlockSpec((1, 1, blk), lambda i: (i, 0, 0)),
                   pl.BlockSpec((1, 1, blk), lambda i: (i, 0, 0))] * 2,
        out_shape=[jax.ShapeDtypeStruct((np_, c), jnp.float32),
                   jax.ShapeDtypeStruct((np_, c), jnp.float32),
                   jax.ShapeDtypeStruct((nb, 1, blk), jnp.float32),
                   jax.ShapeDtypeStruct((nb, 1, blk), jnp.float32)] * 2,
    )(x_pad, wsrc1, wres1, asw1.reshape(1, c), wdv1.reshape(1, din),
      wsrc2, wres2, asw2.reshape(1, c), wdv2.reshape(1, din))
    return out


def _phase_edge_logits(ea_t8, w84):
    _, e = ea_t8.shape
    blk = 2000
    nb = e // blk

    def body(ea_ref, w_ref, o0, o1, o2, o3):
        ea = ea_ref[...][0]
        w = w_ref[...]
        for d, ref in enumerate((o0, o1, o2, o3)):
            ref[0, 0, :] = jnp.sum(ea * w[:, d:d + 1], axis=0)

    out = pl.pallas_call(
        body,
        grid=(nb,),
        in_specs=[pl.BlockSpec((1, 8, blk), lambda i: (i, 0, 0)),
                  pl.BlockSpec((8, 4), lambda i: (0, 0))],
        out_specs=[pl.BlockSpec((1, 1, blk), lambda i: (i, 0, 0))] * 4,
        out_shape=[jax.ShapeDtypeStruct((nb, 1, blk), jnp.float32)] * 4,
    )(ea_t8.reshape(8, nb, blk).transpose(1, 0, 2), w84)
    return [o.reshape(e) for o in out]


def _phase_c(u, den, res, g, b, bias, wsrc, wres, asw, wdv):
    np_, c = u.shape
    blk = 1024
    nb = np_ // blk

    def body(u_ref, den_ref, res_ref, g_ref, b_ref, bias_ref,
             ws_ref, wr_ref, aw_ref, wd_ref,
             hs_ref, res2_ref, as_ref, ad_ref):
        uu = u_ref[...]
        dd = den_ref[0, 0]
        h = uu / (dd + 1e-16)[:, None] + res_ref[...] + bias_ref[...]
        m = jnp.mean(h, axis=-1, keepdims=True)
        v = jnp.mean((h - m) ** 2, axis=-1, keepdims=True)
        h = (h - m) / jnp.sqrt(v + 1e-5) * g_ref[...] + b_ref[...]
        h = jnp.where(h >= 0.0, h, 0.01 * h)
        hs = jnp.dot(h, ws_ref[...], preferred_element_type=jnp.float32)
        hs_ref[...] = hs
        res2_ref[...] = jnp.dot(h, wr_ref[...], preferred_element_type=jnp.float32)
        as_ref[0, 0, :] = jnp.sum(hs * aw_ref[...], axis=1)
        ad_ref[0, 0, :] = jnp.sum(h * wd_ref[...], axis=1)

    full = lambda s: pl.BlockSpec(s, lambda i: (0,) * len(s))
    return pl.pallas_call(
        body,
        grid=(nb,),
        in_specs=[pl.BlockSpec((blk, c), lambda i: (i, 0)),
                  pl.BlockSpec((1, 1, blk), lambda i: (i, 0, 0)),
                  pl.BlockSpec((blk, c), lambda i: (i, 0)),
                  full((1, c)), full((1, c)), full((1, c)),
                  full((c, c)), full((c, c)), full((1, c)), full((1, c))],
        out_specs=[pl.BlockSpec((blk, c), lambda i: (i, 0)),
                   pl.BlockSpec((blk, c), lambda i: (i, 0)),
                   pl.BlockSpec((1, 1, blk), lambda i: (i, 0, 0)),
                   pl.BlockSpec((1, 1, blk), lambda i: (i, 0, 0))],
        out_shape=[jax.ShapeDtypeStruct((np_, c), jnp.float32),
                   jax.ShapeDtypeStruct((np_, c), jnp.float32),
                   jax.ShapeDtypeStruct((nb, 1, blk), jnp.float32),
                   jax.ShapeDtypeStruct((nb, 1, blk), jnp.float32)],
    )(u, den.reshape(nb, 1, blk), res, g.reshape(1, c), b.reshape(1, c),
      bias.reshape(1, c), wsrc, wres, asw.reshape(1, c), wdv.reshape(1, c))


def _phase_e(u1, den1, res1, bias1, u2, den2, res2, bias2):
    np_, c = u1.shape
    blk = 1024
    nb = np_ // blk

    def body(u1_ref, d1_ref, r1_ref, b1_ref, u2_ref, d2_ref, r2_ref, b2_ref,
             out_ref):
        hs = []
        for u_ref, d_ref, r_ref, b_ref in ((u1_ref, d1_ref, r1_ref, b1_ref),
                                           (u2_ref, d2_ref, r2_ref, b2_ref)):
            uu = u_ref[...]
            dd = d_ref[0, 0]
            hs.append(uu / (dd + 1e-16)[:, None] + r_ref[...] + b_ref[...])
        out_ref[...] = jnp.concatenate(hs, axis=1)

    full = lambda s: pl.BlockSpec(s, lambda i: (0,) * len(s))
    per_dir = [pl.BlockSpec((blk, c), lambda i: (i, 0)),
               pl.BlockSpec((1, 1, blk), lambda i: (i, 0, 0)),
               pl.BlockSpec((blk, c), lambda i: (i, 0)),
               full((1, c))]
    return pl.pallas_call(
        body,
        grid=(nb,),
        in_specs=per_dir * 2,
        out_specs=pl.BlockSpec((blk, 2 * c), lambda i: (i, 0)),
        out_shape=jax.ShapeDtypeStruct((np_, 2 * c), jnp.float32),
    )(u1, den1.reshape(nb, 1, blk), res1, bias1.reshape(1, c),
      u2, den2.reshape(nb, 1, blk), res2, bias2.reshape(1, c))


def kernel(task_embedding, edge_attr, params, edge_index):
    n, din = task_embedding.shape
    e, edim = edge_attr.shape
    c = params["dep1"]["att_src"].shape[1]
    np_ = 10240

    x_pad = jnp.pad(task_embedding, ((0, np_ - n), (0, 0)))

    def _vecs(p):
        return (p["att_src"][0],
                p["W_dst"] @ p["att_dst"][0],
                p["W_edge"] @ p["att_edge"][0])

    as1d, wd1d, we1d = _vecs(params["dep1"])
    as1n, wd1n, we1n = _vecs(params["dan1"])
    as2d, wd2d, we2d = _vecs(params["dep2"])
    as2n, wd2n, we2n = _vecs(params["dan2"])

    ea_t8 = jnp.pad(edge_attr.T, ((0, 8 - edim), (0, 0)))
    w84 = jnp.pad(jnp.stack([we1d, we1n, we2d, we2n], axis=1),
                  ((0, 8 - edim), (0, 0)))
    ae1d, ae1n, ae2d, ae2n = _phase_edge_logits(ea_t8, w84)

    nchunk = e // (NS * K)
    src = edge_index[0].astype(jnp.int32)
    dst = edge_index[1].astype(jnp.int32)
    sd = jnp.stack([src + dst * 16384,
                    dst + src * 16384]).reshape(NC, NS, nchunk, K)

    edge_pass = _make_edge_pass(np_, c, e)

    hs1d, res1d, a_s1d, a_d1d, hs1n, res1n, a_s1n, a_d1n = _phase_a(
        x_pad,
        params["dep1"]["W_src"], params["dep1"]["W_res"], as1d, wd1d,
        params["dan1"]["W_src"], params["dan1"]["W_res"], as1n, wd1n)

    u1, den1 = edge_pass(
        jnp.stack([a_s1d.reshape(np_), a_s1n.reshape(np_)]),
        jnp.stack([a_d1d.reshape(np_), a_d1n.reshape(np_)]),
        jnp.stack([ae1d, ae1n]).reshape(NC, NS, nchunk, K),
        jnp.stack([hs1d, hs1n]), sd)

    hs2d, res2d, a_s2d, a_d2d = _phase_c(
        u1[0], den1[0], res1d, params["ln_dep_g"], params["ln_dep_b"],
        params["dep1"]["bias"], params["dep2"]["W_src"],
        params["dep2"]["W_res"], as2d, wd2d)
    hs2n, res2n, a_s2n, a_d2n = _phase_c(
        u1[1], den1[1], res1n, params["ln_dan_g"], params["ln_dan_b"],
        params["dan1"]["bias"], params["dan2"]["W_src"],
        params["dan2"]["W_res"], as2n, wd2n)

    u2, den2 = edge_pass(
        jnp.stack([a_s2d.reshape(np_), a_s2n.reshape(np_)]),
        jnp.stack([a_d2d.reshape(np_), a_d2n.reshape(np_)]),
        jnp.stack([ae2d, ae2n]).reshape(NC, NS, nchunk, K),
        jnp.stack([hs2d, hs2n]), sd)

    out = _phase_e(u2[0], den2[0], res2d, params["dep2"]["bias"],
                   u2[1], den2[1], res2n, params["dan2"]["bias"])
    return out[:n]
